# Initial kernel scaffold; baseline (speedup 1.0000x reference)
import jax
import jax.numpy as jnp
from jax.experimental import pallas as pl


def kernel(x, edge_index, W1, b1, W2, b2, W3, b3, W4, b4, W5, b5):
    raise NotImplementedError("write your pallas kernel here")


# speedup vs baseline: 15.6074x; 15.6074x over previous
import functools

import jax
import jax.numpy as jnp
from jax import lax
from jax.experimental import pallas as pl
from jax.experimental.pallas import tpu as pltpu
from jax.experimental.pallas import tpu_sc as plsc

N_NODES = 100000
N_EDGES = 3200000
F = 16
NC, NS = 2, 16
N_TILES = NC * NS
RB = 8
R_PER_TILE = 800
R_TOTAL = N_TILES * R_PER_TILE
E_PAD = R_TOTAL * 128
N_PAD = 100096
DUMMY = N_NODES

_mesh = plsc.VectorSubcoreMesh(core_axis_name="c", subcore_axis_name="s")


def _sc_pass(table_hbm, srcr_hbm, dstr_hbm, zeros_hbm, out_hbm,
             acc, sbuf, dbuf, rows, sem):
    cid = lax.axis_index("c")
    sid = lax.axis_index("s")
    w = sid * NC + cid

    zrows = N_PAD // NS
    pltpu.sync_copy(zeros_hbm.at[pl.ds(sid * zrows, zrows)],
                    acc.at[pl.ds(sid * zrows, zrows)])
    plsc.subcore_barrier()

    base = w * R_PER_TILE

    @pl.loop(0, R_PER_TILE, step=RB)
    def _(r0):
        pltpu.sync_copy(srcr_hbm.at[pl.ds(base + r0, RB)], sbuf)
        pltpu.sync_copy(dstr_hbm.at[pl.ds(base + r0, RB)], dbuf)
        for j in range(RB):
            pltpu.async_copy(table_hbm.at[sbuf.at[j]],
                             rows.at[pl.ds(j * 128, 128)], sem).wait()
            pltpu.sync_copy(rows.at[pl.ds(j * 128, 128)],
                            acc.at[dbuf.at[j]], add=True)

    plsc.subcore_barrier()
    pltpu.sync_copy(acc.at[pl.ds(sid * zrows, zrows)],
                    out_hbm.at[cid].at[pl.ds(sid * zrows, zrows)])


def _make_sc_pass():
    return functools.partial(
        pl.kernel,
        out_type=jax.ShapeDtypeStruct((NC, N_PAD, F), jnp.float32),
        mesh=_mesh,
        scratch_types=[
            pltpu.VMEM_SHARED((N_PAD, F), jnp.float32),
            pltpu.VMEM((RB, 128), jnp.int32),
            pltpu.VMEM((RB, 128), jnp.int32),
            pltpu.VMEM((RB * 128, F), jnp.float32),
            pltpu.SemaphoreType.DMA,
        ],
        compiler_params=pltpu.CompilerParams(use_tc_tiling_on_sc=False),
    )(_sc_pass)


_BN = 2000


def _dense0_body(x_ref, dp_ref, W1, b1, W2, b2, W3, b3, W4, b4, W5, b5,
                 qt_ref, V_ref):
    C5 = W5[...]
    C4 = jnp.dot(W4[...], C5, preferred_element_type=jnp.float32)
    C3 = jnp.dot(W3[...], C4, preferred_element_type=jnp.float32)
    C2 = jnp.dot(W2[...], C3, preferred_element_type=jnp.float32)
    Wall = jnp.dot(W1[...], C2, preferred_element_type=jnp.float32)
    v1 = jnp.dot(b1[...], C2, preferred_element_type=jnp.float32)
    v2 = jnp.dot(b2[...], C3, preferred_element_type=jnp.float32)
    v3 = jnp.dot(b3[...], C4, preferred_element_type=jnp.float32)
    v4 = jnp.dot(b4[...], C5, preferred_element_type=jnp.float32)
    v5 = b5[...]
    V_ref[...] = jnp.stack([v1, v2, v3, v4, v5])

    p = jnp.dot(x_ref[...], Wall, preferred_element_type=jnp.float32)
    deg = dp_ref[0, :, 0] + dp_ref[1, :, 0] + 1.0
    dinv = lax.rsqrt(deg)
    qt = p * dinv[:, None]
    qt_ref[...] = jnp.concatenate(
        [qt, jnp.zeros((qt.shape[0], F - 3), jnp.float32)], axis=1)


def _dense_mid_body(k, up_ref, qt_ref, dp_ref, V_ref, out_ref):
    t = up_ref[0] + up_ref[1] + qt_ref[...]
    deg = dp_ref[0, :, 0] + dp_ref[1, :, 0] + 1.0
    dinv = lax.rsqrt(deg)
    vk = V_ref[k - 1]
    vk16 = jnp.concatenate([vk, jnp.zeros((F - 3,), jnp.float32)])
    out_ref[...] = (dinv * dinv)[:, None] * t + dinv[:, None] * vk16[None, :]


def _dense_final_body(up_ref, qt_ref, dp_ref, V_ref, out_ref):
    t = up_ref[0] + up_ref[1] + qt_ref[...]
    deg = dp_ref[0, :, 0] + dp_ref[1, :, 0] + 1.0
    dinv = lax.rsqrt(deg)
    out_ref[...] = dinv[:, None] * t[:, :3] + V_ref[4][None, :]


def _full(shape):
    return pl.BlockSpec(shape, lambda i: (0,) * len(shape))


_ROWBLK = pl.BlockSpec((_BN, F), lambda i: (i, 0))
_DPBLK = pl.BlockSpec((NC, _BN, F), lambda i: (0, i, 0))
_GRID = (N_NODES // _BN,)


def kernel(x, edge_index, W1, b1, W2, b2, W3, b3, W4, b4, W5, b5):
    src = edge_index[0].astype(jnp.int32)
    dst = edge_index[1].astype(jnp.int32)
    pad = jnp.full((E_PAD - N_EDGES,), DUMMY, dtype=jnp.int32)
    src_r = jnp.concatenate([src, pad]).reshape(R_TOTAL, 128)
    dst_r = jnp.concatenate([dst, pad]).reshape(R_TOTAL, 128)
    zeros_pad = jnp.zeros((N_PAD, F), jnp.float32)
    ones_tab = jnp.ones((N_PAD, F), jnp.float32)

    sc_pass = _make_sc_pass()

    dp = sc_pass(ones_tab, src_r, dst_r, zeros_pad)

    qt, V = pl.pallas_call(
        _dense0_body,
        grid=_GRID,
        in_specs=[pl.BlockSpec((_BN, 8), lambda i: (i, 0)), _DPBLK,
                  _full((8, 32)), _full((32,)),
                  _full((32, 32)), _full((32,)),
                  _full((32, 32)), _full((32,)),
                  _full((32, 32)), _full((32,)),
                  _full((32, 3)), _full((3,))],
        out_specs=[_ROWBLK, _full((5, 3))],
        out_shape=[jax.ShapeDtypeStruct((N_PAD, F), jnp.float32),
                   jax.ShapeDtypeStruct((5, 3), jnp.float32)],
    )(x, dp, W1, b1, W2, b2, W3, b3, W4, b4, W5, b5)

    for k in (1, 2, 3, 4):
        up = sc_pass(qt, src_r, dst_r, zeros_pad)
        qt = pl.pallas_call(
            functools.partial(_dense_mid_body, k),
            grid=_GRID,
            in_specs=[_DPBLK, _ROWBLK, _DPBLK, _full((5, 3))],
            out_specs=_ROWBLK,
            out_shape=jax.ShapeDtypeStruct((N_PAD, F), jnp.float32),
        )(up, qt, dp, V)

    up = sc_pass(qt, src_r, dst_r, zeros_pad)
    out = pl.pallas_call(
        _dense_final_body,
        grid=_GRID,
        in_specs=[_DPBLK, _ROWBLK, _DPBLK, _full((5, 3))],
        out_specs=pl.BlockSpec((_BN, 3), lambda i: (i, 0)),
        out_shape=jax.ShapeDtypeStruct((N_NODES, 3), jnp.float32),
    )(up, qt, dp, V)
    return out


# speedup vs baseline: 30.5899x; 1.9600x over previous
import functools

import jax
import jax.numpy as jnp
from jax import lax
from jax.experimental import pallas as pl
from jax.experimental.pallas import tpu as pltpu
from jax.experimental.pallas import tpu_sc as plsc

N_NODES = 100000
N_EDGES = 3200000
F = 8
NC, NS = 2, 16
N_TILES = NC * NS
RB = 8
R_PER_TILE = 800
R_TOTAL = N_TILES * R_PER_TILE
E_PAD = R_TOTAL * 128
N_PAD = 100096
DUMMY = N_NODES

_mesh = plsc.VectorSubcoreMesh(core_axis_name="c", subcore_axis_name="s")


KB = 4


def _sc_pass(do_gather, table_hbm, srcr_hbm, dstr_hbm, zeros_hbm, out_hbm,
             acc, sbuf, dbuf, rows_a, rows_b, semg, sems):
    cid = lax.axis_index("c")
    sid = lax.axis_index("s")
    w = sid * NC + cid

    zrows = N_PAD // NS
    pltpu.sync_copy(zeros_hbm.at[pl.ds(sid * zrows, zrows)],
                    acc.at[pl.ds(sid * zrows, zrows)])
    if not do_gather:
        pltpu.sync_copy(table_hbm.at[pl.ds(0, RB * 128)], rows_a)
        pltpu.sync_copy(table_hbm.at[pl.ds(0, RB * 128)], rows_b)
    plsc.subcore_barrier()

    base = w * R_PER_TILE

    @pl.loop(0, R_PER_TILE, step=KB * RB)
    def _(r0):
        if do_gather:
            pltpu.sync_copy(srcr_hbm.at[pl.ds(base + r0, KB * RB)], sbuf)
        pltpu.sync_copy(dstr_hbm.at[pl.ds(base + r0, KB * RB)], dbuf)
        pending = [None, None]
        for k in range(KB):
            rows = rows_a if k % 2 == 0 else rows_b
            if pending[k % 2] is not None:
                for d in pending[k % 2]:
                    d.wait()
            if do_gather:
                gs = [pltpu.async_copy(table_hbm.at[sbuf.at[k * RB + j]],
                                       rows.at[pl.ds(j * 128, 128)], semg)
                      for j in range(RB)]
                for d in gs:
                    d.wait()
            pending[k % 2] = [
                pltpu.async_copy(rows.at[pl.ds(j * 128, 128)],
                                 acc.at[dbuf.at[k * RB + j]], sems, add=True)
                for j in range(RB)]
        for ds_ in pending:
            if ds_ is not None:
                for d in ds_:
                    d.wait()

    plsc.subcore_barrier()
    pltpu.sync_copy(acc.at[pl.ds(sid * zrows, zrows)],
                    out_hbm.at[cid].at[pl.ds(sid * zrows, zrows)])


def _make_sc_pass(do_gather):
    return functools.partial(
        pl.kernel,
        out_type=jax.ShapeDtypeStruct((NC, N_PAD, F), jnp.float32),
        mesh=_mesh,
        scratch_types=[
            pltpu.VMEM_SHARED((N_PAD, F), jnp.float32),
            pltpu.VMEM((KB * RB, 128), jnp.int32),
            pltpu.VMEM((KB * RB, 128), jnp.int32),
            pltpu.VMEM((RB * 128, F), jnp.float32),
            pltpu.VMEM((RB * 128, F), jnp.float32),
            pltpu.SemaphoreType.DMA,
            pltpu.SemaphoreType.DMA,
        ],
        compiler_params=pltpu.CompilerParams(use_tc_tiling_on_sc=False),
    )(functools.partial(_sc_pass, do_gather))


_BN = 2000


def _rsqrt(x):
    y = lax.rsqrt(x)
    y = y * (1.5 - 0.5 * x * y * y)
    y = y * (1.5 - 0.5 * x * y * y)
    return y


def _dense0_body(x_ref, dp_ref, W1, b1, W2, b2, W3, b3, W4, b4, W5, b5,
                 qt_ref, V_ref):
    C5 = W5[...]
    C4 = jnp.dot(W4[...], C5, preferred_element_type=jnp.float32, precision=lax.Precision.HIGHEST)
    C3 = jnp.dot(W3[...], C4, preferred_element_type=jnp.float32, precision=lax.Precision.HIGHEST)
    C2 = jnp.dot(W2[...], C3, preferred_element_type=jnp.float32, precision=lax.Precision.HIGHEST)
    Wall = jnp.dot(W1[...], C2, preferred_element_type=jnp.float32, precision=lax.Precision.HIGHEST)
    v1 = jnp.dot(b1[...], C2, preferred_element_type=jnp.float32, precision=lax.Precision.HIGHEST)
    v2 = jnp.dot(b2[...], C3, preferred_element_type=jnp.float32, precision=lax.Precision.HIGHEST)
    v3 = jnp.dot(b3[...], C4, preferred_element_type=jnp.float32, precision=lax.Precision.HIGHEST)
    v4 = jnp.dot(b4[...], C5, preferred_element_type=jnp.float32, precision=lax.Precision.HIGHEST)
    v5 = b5[...]
    V_ref[...] = jnp.stack([v1, v2, v3, v4, v5])

    p = jnp.dot(x_ref[...], Wall, preferred_element_type=jnp.float32, precision=lax.Precision.HIGHEST)
    deg = dp_ref[0, :, 0] + dp_ref[1, :, 0] + 1.0
    dinv = _rsqrt(deg)
    qt = p * dinv[:, None]
    qt_ref[...] = jnp.concatenate(
        [qt, jnp.zeros((qt.shape[0], F - 3), jnp.float32)], axis=1)


def _dense_mid_body(k, up_ref, qt_ref, dp_ref, V_ref, out_ref):
    t = up_ref[0] + up_ref[1] + qt_ref[...]
    deg = dp_ref[0, :, 0] + dp_ref[1, :, 0] + 1.0
    dinv = _rsqrt(deg)
    vk = V_ref[k - 1]
    vk16 = jnp.concatenate([vk, jnp.zeros((F - 3,), jnp.float32)])
    out_ref[...] = (dinv * dinv)[:, None] * t + dinv[:, None] * vk16[None, :]


def _dense_final_body(up_ref, qt_ref, dp_ref, V_ref, out_ref):
    t = up_ref[0] + up_ref[1] + qt_ref[...]
    deg = dp_ref[0, :, 0] + dp_ref[1, :, 0] + 1.0
    dinv = _rsqrt(deg)
    out_ref[...] = dinv[:, None] * t[:, :3] + V_ref[4][None, :]


def _full(shape):
    return pl.BlockSpec(shape, lambda i: (0,) * len(shape))


_ROWBLK = pl.BlockSpec((_BN, F), lambda i: (i, 0))
_DPBLK = pl.BlockSpec((NC, _BN, F), lambda i: (0, i, 0))
_GRID = (N_NODES // _BN,)


def kernel(x, edge_index, W1, b1, W2, b2, W3, b3, W4, b4, W5, b5):
    src = edge_index[0].astype(jnp.int32)
    dst = edge_index[1].astype(jnp.int32)
    pad = jnp.full((E_PAD - N_EDGES,), DUMMY, dtype=jnp.int32)
    src_r = jnp.concatenate([src, pad]).reshape(R_TOTAL, 128)
    dst_r = jnp.concatenate([dst, pad]).reshape(R_TOTAL, 128)
    zeros_pad = jnp.zeros((N_PAD, F), jnp.float32)
    ones_tab = jnp.ones((N_PAD, F), jnp.float32)

    sc_pass = _make_sc_pass(True)
    sc_count = _make_sc_pass(False)

    dp = sc_count(ones_tab, src_r, dst_r, zeros_pad)

    qt, V = pl.pallas_call(
        _dense0_body,
        grid=_GRID,
        in_specs=[pl.BlockSpec((_BN, 8), lambda i: (i, 0)), _DPBLK,
                  _full((8, 32)), _full((32,)),
                  _full((32, 32)), _full((32,)),
                  _full((32, 32)), _full((32,)),
                  _full((32, 32)), _full((32,)),
                  _full((32, 3)), _full((3,))],
        out_specs=[_ROWBLK, _full((5, 3))],
        out_shape=[jax.ShapeDtypeStruct((N_PAD, F), jnp.float32),
                   jax.ShapeDtypeStruct((5, 3), jnp.float32)],
    )(x, dp, W1, b1, W2, b2, W3, b3, W4, b4, W5, b5)

    for k in (1, 2, 3, 4):
        up = sc_pass(qt, src_r, dst_r, zeros_pad)
        qt = pl.pallas_call(
            functools.partial(_dense_mid_body, k),
            grid=_GRID,
            in_specs=[_DPBLK, _ROWBLK, _DPBLK, _full((5, 3))],
            out_specs=_ROWBLK,
            out_shape=jax.ShapeDtypeStruct((N_PAD, F), jnp.float32),
        )(up, qt, dp, V)

    up = sc_pass(qt, src_r, dst_r, zeros_pad)
    out = pl.pallas_call(
        _dense_final_body,
        grid=_GRID,
        in_specs=[_DPBLK, _ROWBLK, _DPBLK, _full((5, 3))],
        out_specs=pl.BlockSpec((_BN, 3), lambda i: (i, 0)),
        out_shape=jax.ShapeDtypeStruct((N_NODES, 3), jnp.float32),
    )(up, qt, dp, V)
    return out


# speedup vs baseline: 52.4532x; 1.7147x over previous
import functools

import jax
import jax.numpy as jnp
from jax import lax
from jax.experimental import pallas as pl
from jax.experimental.pallas import tpu as pltpu
from jax.experimental.pallas import tpu_sc as plsc

N_NODES = 100000
N_EDGES = 3200000
F = 8
NC, NS = 2, 16
N_TILES = NC * NS
RB = 8
R_PER_TILE = 800
R_TOTAL = N_TILES * R_PER_TILE
E_PAD = R_TOTAL * 128
N_PAD = 100096
DUMMY = N_NODES

_mesh = plsc.VectorSubcoreMesh(core_axis_name="c", subcore_axis_name="s")


KB = 4


def _sc_pass(do_gather, table_hbm, srcr_hbm, dstr_hbm, zeros_hbm, out_hbm,
             acc, tab, sbuf, dbuf, rows_a, rows_b, semg, sems):
    cid = lax.axis_index("c")
    sid = lax.axis_index("s")
    w = sid * NC + cid

    zrows = N_PAD // NS
    pltpu.sync_copy(zeros_hbm.at[pl.ds(sid * zrows, zrows)],
                    acc.at[pl.ds(sid * zrows, zrows)])
    if do_gather:
        pltpu.sync_copy(table_hbm.at[pl.ds(sid * zrows, zrows)],
                        tab.at[pl.ds(sid * zrows, zrows)])
    if not do_gather:
        pltpu.sync_copy(table_hbm.at[pl.ds(0, RB * 128)], rows_a)
        pltpu.sync_copy(table_hbm.at[pl.ds(0, RB * 128)], rows_b)
    plsc.subcore_barrier()

    base = w * R_PER_TILE

    @pl.loop(0, R_PER_TILE, step=KB * RB)
    def _(r0):
        if do_gather:
            pltpu.sync_copy(srcr_hbm.at[pl.ds(base + r0, KB * RB)], sbuf)
        pltpu.sync_copy(dstr_hbm.at[pl.ds(base + r0, KB * RB)], dbuf)
        pending = [None, None]
        for k in range(KB):
            rows = rows_a if k % 2 == 0 else rows_b
            if pending[k % 2] is not None:
                for d in pending[k % 2]:
                    d.wait()
            if do_gather:
                gs = [pltpu.async_copy(tab.at[sbuf.at[k * RB + j]],
                                       rows.at[pl.ds(j * 128, 128)], semg)
                      for j in range(RB)]
                for d in gs:
                    d.wait()
            pending[k % 2] = [
                pltpu.async_copy(rows.at[pl.ds(j * 128, 128)],
                                 acc.at[dbuf.at[k * RB + j]], sems, add=True)
                for j in range(RB)]
        for ds_ in pending:
            if ds_ is not None:
                for d in ds_:
                    d.wait()

    plsc.subcore_barrier()
    pltpu.sync_copy(acc.at[pl.ds(sid * zrows, zrows)],
                    out_hbm.at[cid].at[pl.ds(sid * zrows, zrows)])


def _make_sc_pass(do_gather):
    return functools.partial(
        pl.kernel,
        out_type=jax.ShapeDtypeStruct((NC, N_PAD, F), jnp.float32),
        mesh=_mesh,
        scratch_types=[
            pltpu.VMEM_SHARED((N_PAD, F), jnp.float32),
            pltpu.VMEM_SHARED((N_PAD, F), jnp.float32),
            pltpu.VMEM((KB * RB, 128), jnp.int32),
            pltpu.VMEM((KB * RB, 128), jnp.int32),
            pltpu.VMEM((RB * 128, F), jnp.float32),
            pltpu.VMEM((RB * 128, F), jnp.float32),
            pltpu.SemaphoreType.DMA,
            pltpu.SemaphoreType.DMA,
        ],
        compiler_params=pltpu.CompilerParams(use_tc_tiling_on_sc=False),
    )(functools.partial(_sc_pass, do_gather))


_BN = 2000


def _rsqrt(x):
    y = lax.rsqrt(x)
    y = y * (1.5 - 0.5 * x * y * y)
    y = y * (1.5 - 0.5 * x * y * y)
    return y


def _dense0_body(x_ref, dp_ref, W1, b1, W2, b2, W3, b3, W4, b4, W5, b5,
                 qt_ref, V_ref):
    C5 = W5[...]
    C4 = jnp.dot(W4[...], C5, preferred_element_type=jnp.float32, precision=lax.Precision.HIGHEST)
    C3 = jnp.dot(W3[...], C4, preferred_element_type=jnp.float32, precision=lax.Precision.HIGHEST)
    C2 = jnp.dot(W2[...], C3, preferred_element_type=jnp.float32, precision=lax.Precision.HIGHEST)
    Wall = jnp.dot(W1[...], C2, preferred_element_type=jnp.float32, precision=lax.Precision.HIGHEST)
    v1 = jnp.dot(b1[...], C2, preferred_element_type=jnp.float32, precision=lax.Precision.HIGHEST)
    v2 = jnp.dot(b2[...], C3, preferred_element_type=jnp.float32, precision=lax.Precision.HIGHEST)
    v3 = jnp.dot(b3[...], C4, preferred_element_type=jnp.float32, precision=lax.Precision.HIGHEST)
    v4 = jnp.dot(b4[...], C5, preferred_element_type=jnp.float32, precision=lax.Precision.HIGHEST)
    v5 = b5[...]
    V_ref[...] = jnp.stack([v1, v2, v3, v4, v5])

    p = jnp.dot(x_ref[...], Wall, preferred_element_type=jnp.float32, precision=lax.Precision.HIGHEST)
    deg = dp_ref[0, :, 0] + dp_ref[1, :, 0] + 1.0
    dinv = _rsqrt(deg)
    qt = p * dinv[:, None]
    qt_ref[...] = jnp.concatenate(
        [qt, jnp.zeros((qt.shape[0], F - 3), jnp.float32)], axis=1)


def _dense_mid_body(k, up_ref, qt_ref, dp_ref, V_ref, out_ref):
    t = up_ref[0] + up_ref[1] + qt_ref[...]
    deg = dp_ref[0, :, 0] + dp_ref[1, :, 0] + 1.0
    dinv = _rsqrt(deg)
    vk = V_ref[k - 1]
    vk16 = jnp.concatenate([vk, jnp.zeros((F - 3,), jnp.float32)])
    out_ref[...] = (dinv * dinv)[:, None] * t + dinv[:, None] * vk16[None, :]


def _dense_final_body(up_ref, qt_ref, dp_ref, V_ref, out_ref):
    t = up_ref[0] + up_ref[1] + qt_ref[...]
    deg = dp_ref[0, :, 0] + dp_ref[1, :, 0] + 1.0
    dinv = _rsqrt(deg)
    out_ref[...] = dinv[:, None] * t[:, :3] + V_ref[4][None, :]


def _full(shape):
    return pl.BlockSpec(shape, lambda i: (0,) * len(shape))


_ROWBLK = pl.BlockSpec((_BN, F), lambda i: (i, 0))
_DPBLK = pl.BlockSpec((NC, _BN, F), lambda i: (0, i, 0))
_GRID = (N_NODES // _BN,)


def kernel(x, edge_index, W1, b1, W2, b2, W3, b3, W4, b4, W5, b5):
    src = edge_index[0].astype(jnp.int32)
    dst = edge_index[1].astype(jnp.int32)
    pad = jnp.full((E_PAD - N_EDGES,), DUMMY, dtype=jnp.int32)
    src_r = jnp.concatenate([src, pad]).reshape(R_TOTAL, 128)
    dst_r = jnp.concatenate([dst, pad]).reshape(R_TOTAL, 128)
    zeros_pad = jnp.zeros((N_PAD, F), jnp.float32)
    ones_tab = jnp.ones((N_PAD, F), jnp.float32)

    sc_pass = _make_sc_pass(True)
    sc_count = _make_sc_pass(False)

    dp = sc_count(ones_tab, src_r, dst_r, zeros_pad)

    qt, V = pl.pallas_call(
        _dense0_body,
        grid=_GRID,
        in_specs=[pl.BlockSpec((_BN, 8), lambda i: (i, 0)), _DPBLK,
                  _full((8, 32)), _full((32,)),
                  _full((32, 32)), _full((32,)),
                  _full((32, 32)), _full((32,)),
                  _full((32, 32)), _full((32,)),
                  _full((32, 3)), _full((3,))],
        out_specs=[_ROWBLK, _full((5, 3))],
        out_shape=[jax.ShapeDtypeStruct((N_PAD, F), jnp.float32),
                   jax.ShapeDtypeStruct((5, 3), jnp.float32)],
    )(x, dp, W1, b1, W2, b2, W3, b3, W4, b4, W5, b5)

    for k in (1, 2, 3, 4):
        up = sc_pass(qt, src_r, dst_r, zeros_pad)
        qt = pl.pallas_call(
            functools.partial(_dense_mid_body, k),
            grid=_GRID,
            in_specs=[_DPBLK, _ROWBLK, _DPBLK, _full((5, 3))],
            out_specs=_ROWBLK,
            out_shape=jax.ShapeDtypeStruct((N_PAD, F), jnp.float32),
        )(up, qt, dp, V)

    up = sc_pass(qt, src_r, dst_r, zeros_pad)
    out = pl.pallas_call(
        _dense_final_body,
        grid=_GRID,
        in_specs=[_DPBLK, _ROWBLK, _DPBLK, _full((5, 3))],
        out_specs=pl.BlockSpec((_BN, 3), lambda i: (i, 0)),
        out_shape=jax.ShapeDtypeStruct((N_NODES, 3), jnp.float32),
    )(up, qt, dp, V)
    return out


# speedup vs baseline: 53.1412x; 1.0131x over previous
import functools

import jax
import jax.numpy as jnp
from jax import lax
from jax.experimental import pallas as pl
from jax.experimental.pallas import tpu as pltpu
from jax.experimental.pallas import tpu_sc as plsc

N_NODES = 100000
N_EDGES = 3200000
F = 8
NC, NS = 2, 16
N_TILES = NC * NS
RB = 8
R_PER_TILE = 800
R_TOTAL = N_TILES * R_PER_TILE
E_PAD = R_TOTAL * 128
N_PAD = 100096
DUMMY = N_NODES

_mesh = plsc.VectorSubcoreMesh(core_axis_name="c", subcore_axis_name="s")


KB = 5


def _sc_pass(do_gather, table_hbm, srcr_hbm, dstr_hbm, zeros_hbm, out_hbm,
             acc, tab, sbuf, dbuf, rows_a, rows_b, semg, sems):
    cid = lax.axis_index("c")
    sid = lax.axis_index("s")
    w = sid * NC + cid

    zrows = N_PAD // NS
    pltpu.sync_copy(zeros_hbm.at[pl.ds(sid * zrows, zrows)],
                    acc.at[pl.ds(sid * zrows, zrows)])
    if do_gather:
        pltpu.sync_copy(table_hbm.at[pl.ds(sid * zrows, zrows)],
                        tab.at[pl.ds(sid * zrows, zrows)])
    if not do_gather:
        pltpu.sync_copy(table_hbm.at[pl.ds(0, RB * 128)], rows_a)
        pltpu.sync_copy(table_hbm.at[pl.ds(0, RB * 128)], rows_b)
    plsc.subcore_barrier()

    base = w * R_PER_TILE

    @pl.loop(0, R_PER_TILE, step=KB * RB)
    def _(r0):
        if do_gather:
            pltpu.sync_copy(srcr_hbm.at[pl.ds(base + r0, KB * RB)], sbuf)
        pltpu.sync_copy(dstr_hbm.at[pl.ds(base + r0, KB * RB)], dbuf)
        pending = [None, None]
        for k in range(KB):
            rows = rows_a if k % 2 == 0 else rows_b
            if pending[k % 2] is not None:
                for d in pending[k % 2]:
                    d.wait()
            if do_gather:
                gs = [pltpu.async_copy(tab.at[sbuf.at[k * RB + j]],
                                       rows.at[pl.ds(j * 128, 128)], semg)
                      for j in range(RB)]
                for d in gs:
                    d.wait()
            pending[k % 2] = [
                pltpu.async_copy(rows.at[pl.ds(j * 128, 128)],
                                 acc.at[dbuf.at[k * RB + j]], sems, add=True)
                for j in range(RB)]
        for ds_ in pending:
            if ds_ is not None:
                for d in ds_:
                    d.wait()

    plsc.subcore_barrier()
    pltpu.sync_copy(acc.at[pl.ds(sid * zrows, zrows)],
                    out_hbm.at[cid].at[pl.ds(sid * zrows, zrows)])


def _make_sc_pass(do_gather):
    return functools.partial(
        pl.kernel,
        out_type=jax.ShapeDtypeStruct((NC, N_PAD, F), jnp.float32),
        mesh=_mesh,
        scratch_types=[
            pltpu.VMEM_SHARED((N_PAD, F), jnp.float32),
            pltpu.VMEM_SHARED((N_PAD, F), jnp.float32),
            pltpu.VMEM((KB * RB, 128), jnp.int32),
            pltpu.VMEM((KB * RB, 128), jnp.int32),
            pltpu.VMEM((RB * 128, F), jnp.float32),
            pltpu.VMEM((RB * 128, F), jnp.float32),
            pltpu.SemaphoreType.DMA,
            pltpu.SemaphoreType.DMA,
        ],
        compiler_params=pltpu.CompilerParams(use_tc_tiling_on_sc=False),
    )(functools.partial(_sc_pass, do_gather))


_BN = 2000


def _rsqrt(x):
    y = lax.rsqrt(x)
    y = y * (1.5 - 0.5 * x * y * y)
    y = y * (1.5 - 0.5 * x * y * y)
    return y


def _dense0_body(x_ref, dp_ref, W1, b1, W2, b2, W3, b3, W4, b4, W5, b5,
                 qt_ref, V_ref):
    C5 = W5[...]
    C4 = jnp.dot(W4[...], C5, preferred_element_type=jnp.float32, precision=lax.Precision.HIGHEST)
    C3 = jnp.dot(W3[...], C4, preferred_element_type=jnp.float32, precision=lax.Precision.HIGHEST)
    C2 = jnp.dot(W2[...], C3, preferred_element_type=jnp.float32, precision=lax.Precision.HIGHEST)
    Wall = jnp.dot(W1[...], C2, preferred_element_type=jnp.float32, precision=lax.Precision.HIGHEST)
    v1 = jnp.dot(b1[...], C2, preferred_element_type=jnp.float32, precision=lax.Precision.HIGHEST)
    v2 = jnp.dot(b2[...], C3, preferred_element_type=jnp.float32, precision=lax.Precision.HIGHEST)
    v3 = jnp.dot(b3[...], C4, preferred_element_type=jnp.float32, precision=lax.Precision.HIGHEST)
    v4 = jnp.dot(b4[...], C5, preferred_element_type=jnp.float32, precision=lax.Precision.HIGHEST)
    v5 = b5[...]
    V_ref[...] = jnp.stack([v1, v2, v3, v4, v5])

    p = jnp.dot(x_ref[...], Wall, preferred_element_type=jnp.float32, precision=lax.Precision.HIGHEST)
    deg = dp_ref[0, :, 0] + dp_ref[1, :, 0] + 1.0
    dinv = _rsqrt(deg)
    qt = p * dinv[:, None]
    qt_ref[...] = jnp.concatenate(
        [qt, jnp.zeros((qt.shape[0], F - 3), jnp.float32)], axis=1)


def _dense_mid_body(k, up_ref, qt_ref, dp_ref, V_ref, out_ref):
    t = up_ref[0] + up_ref[1] + qt_ref[...]
    deg = dp_ref[0, :, 0] + dp_ref[1, :, 0] + 1.0
    dinv = _rsqrt(deg)
    vk = V_ref[k - 1]
    vk16 = jnp.concatenate([vk, jnp.zeros((F - 3,), jnp.float32)])
    out_ref[...] = (dinv * dinv)[:, None] * t + dinv[:, None] * vk16[None, :]


def _dense_final_body(up_ref, qt_ref, dp_ref, V_ref, out_ref):
    t = up_ref[0] + up_ref[1] + qt_ref[...]
    deg = dp_ref[0, :, 0] + dp_ref[1, :, 0] + 1.0
    dinv = _rsqrt(deg)
    out_ref[...] = dinv[:, None] * t[:, :3] + V_ref[4][None, :]


def _full(shape):
    return pl.BlockSpec(shape, lambda i: (0,) * len(shape))


_ROWBLK = pl.BlockSpec((_BN, F), lambda i: (i, 0))
_DPBLK = pl.BlockSpec((NC, _BN, F), lambda i: (0, i, 0))
_GRID = (N_NODES // _BN,)


def kernel(x, edge_index, W1, b1, W2, b2, W3, b3, W4, b4, W5, b5):
    src = edge_index[0].astype(jnp.int32)
    dst = edge_index[1].astype(jnp.int32)
    pad = jnp.full((E_PAD - N_EDGES,), DUMMY, dtype=jnp.int32)
    src_r = jnp.concatenate([src, pad]).reshape(R_TOTAL, 128)
    dst_r = jnp.concatenate([dst, pad]).reshape(R_TOTAL, 128)
    zeros_pad = jnp.zeros((N_PAD, F), jnp.float32)
    ones_tab = jnp.ones((N_PAD, F), jnp.float32)

    sc_pass = _make_sc_pass(True)
    sc_count = _make_sc_pass(False)

    dp = sc_count(ones_tab, src_r, dst_r, zeros_pad)

    qt, V = pl.pallas_call(
        _dense0_body,
        grid=_GRID,
        in_specs=[pl.BlockSpec((_BN, 8), lambda i: (i, 0)), _DPBLK,
                  _full((8, 32)), _full((32,)),
                  _full((32, 32)), _full((32,)),
                  _full((32, 32)), _full((32,)),
                  _full((32, 32)), _full((32,)),
                  _full((32, 3)), _full((3,))],
        out_specs=[_ROWBLK, _full((5, 3))],
        out_shape=[jax.ShapeDtypeStruct((N_PAD, F), jnp.float32),
                   jax.ShapeDtypeStruct((5, 3), jnp.float32)],
    )(x, dp, W1, b1, W2, b2, W3, b3, W4, b4, W5, b5)

    for k in (1, 2, 3, 4):
        up = sc_pass(qt, src_r, dst_r, zeros_pad)
        qt = pl.pallas_call(
            functools.partial(_dense_mid_body, k),
            grid=_GRID,
            in_specs=[_DPBLK, _ROWBLK, _DPBLK, _full((5, 3))],
            out_specs=_ROWBLK,
            out_shape=jax.ShapeDtypeStruct((N_PAD, F), jnp.float32),
        )(up, qt, dp, V)

    up = sc_pass(qt, src_r, dst_r, zeros_pad)
    out = pl.pallas_call(
        _dense_final_body,
        grid=_GRID,
        in_specs=[_DPBLK, _ROWBLK, _DPBLK, _full((5, 3))],
        out_specs=pl.BlockSpec((_BN, 3), lambda i: (i, 0)),
        out_shape=jax.ShapeDtypeStruct((N_NODES, 3), jnp.float32),
    )(up, qt, dp, V)
    return out


# speedup vs baseline: 77.9992x; 1.4678x over previous
import functools

import jax
import jax.numpy as jnp
from jax import lax
from jax.experimental import pallas as pl
from jax.experimental.pallas import tpu as pltpu
from jax.experimental.pallas import tpu_sc as plsc

N_NODES = 100000
N_EDGES = 3200000
F = 8
NC, NS = 2, 16
N_TILES = NC * NS
RB = 8
R_PER_TILE = 800
R_TOTAL = N_TILES * R_PER_TILE
E_PAD = R_TOTAL * 128
N_PAD = 100096
DUMMY = N_NODES

_mesh = plsc.VectorSubcoreMesh(core_axis_name="c", subcore_axis_name="s")


KB = 5


def _sc_pass(do_gather, table_hbm, srcr_hbm, dstr_hbm, zeros_hbm, out_hbm,
             acc, tab, sbuf, dbuf, rows_a, rows_b, semg, sems):
    cid = lax.axis_index("c")
    sid = lax.axis_index("s")
    w = sid * NC + cid

    zrows = N_PAD // NS
    pltpu.sync_copy(zeros_hbm.at[pl.ds(sid * zrows, zrows)],
                    acc.at[pl.ds(sid * zrows, zrows)])
    if do_gather:
        pltpu.sync_copy(table_hbm.at[pl.ds(sid * zrows, zrows)],
                        tab.at[pl.ds(sid * zrows, zrows)])
    if not do_gather:
        pltpu.sync_copy(table_hbm.at[pl.ds(0, RB * 128)], rows_a)
        pltpu.sync_copy(table_hbm.at[pl.ds(0, RB * 128)], rows_b)
    plsc.subcore_barrier()

    base = w * R_PER_TILE

    @pl.loop(0, R_PER_TILE, step=KB * RB)
    def _(r0):
        if do_gather:
            pltpu.sync_copy(srcr_hbm.at[pl.ds(base + r0, KB * RB)], sbuf)
        pltpu.sync_copy(dstr_hbm.at[pl.ds(base + r0, KB * RB)], dbuf)
        pending = [None, None]
        for k in range(KB):
            rows = rows_a if k % 2 == 0 else rows_b
            if pending[k % 2] is not None:
                for d in pending[k % 2]:
                    d.wait()
            if do_gather:
                gs = [pltpu.async_copy(tab.at[sbuf.at[k * RB + j]],
                                       rows.at[pl.ds(j * 128, 128)], semg)
                      for j in range(RB)]
                for d in gs:
                    d.wait()
            pending[k % 2] = [
                pltpu.async_copy(rows.at[pl.ds(j * 128, 128)],
                                 acc.at[dbuf.at[k * RB + j]], sems, add=True)
                for j in range(RB)]
        for ds_ in pending:
            if ds_ is not None:
                for d in ds_:
                    d.wait()

    plsc.subcore_barrier()
    pltpu.sync_copy(acc.at[pl.ds(sid * zrows, zrows)],
                    out_hbm.at[cid].at[pl.ds(sid * zrows, zrows)])


def _make_sc_pass(do_gather):
    return functools.partial(
        pl.kernel,
        out_type=jax.ShapeDtypeStruct((NC, N_PAD, F), jnp.float32),
        mesh=_mesh,
        scratch_types=[
            pltpu.VMEM_SHARED((N_PAD, F), jnp.float32),
            pltpu.VMEM_SHARED((N_PAD, F), jnp.float32),
            pltpu.VMEM((KB * RB, 128), jnp.int32),
            pltpu.VMEM((KB * RB, 128), jnp.int32),
            pltpu.VMEM((RB * 128, F), jnp.float32),
            pltpu.VMEM((RB * 128, F), jnp.float32),
            pltpu.SemaphoreType.DMA,
            pltpu.SemaphoreType.DMA,
        ],
        compiler_params=pltpu.CompilerParams(use_tc_tiling_on_sc=False),
    )(functools.partial(_sc_pass, do_gather))


_BN = 2000


def _rsqrt(x):
    y = lax.rsqrt(x)
    y = y * (1.5 - 0.5 * x * y * y)
    y = y * (1.5 - 0.5 * x * y * y)
    return y


def _dense0_body(x_ref, dp_ref, W1, b1, W2, b2, W3, b3, W4, b4, W5, b5,
                 qt_ref, V_ref):
    C5 = W5[...]
    C4 = jnp.dot(W4[...], C5, preferred_element_type=jnp.float32, precision=lax.Precision.HIGHEST)
    C3 = jnp.dot(W3[...], C4, preferred_element_type=jnp.float32, precision=lax.Precision.HIGHEST)
    C2 = jnp.dot(W2[...], C3, preferred_element_type=jnp.float32, precision=lax.Precision.HIGHEST)
    Wall = jnp.dot(W1[...], C2, preferred_element_type=jnp.float32, precision=lax.Precision.HIGHEST)
    v1 = jnp.dot(b1[...], C2, preferred_element_type=jnp.float32, precision=lax.Precision.HIGHEST)
    v2 = jnp.dot(b2[...], C3, preferred_element_type=jnp.float32, precision=lax.Precision.HIGHEST)
    v3 = jnp.dot(b3[...], C4, preferred_element_type=jnp.float32, precision=lax.Precision.HIGHEST)
    v4 = jnp.dot(b4[...], C5, preferred_element_type=jnp.float32, precision=lax.Precision.HIGHEST)
    v5 = b5[...]
    V_ref[...] = jnp.stack([v1, v2, v3, v4, v5])

    p = jnp.dot(x_ref[...], Wall, preferred_element_type=jnp.float32, precision=lax.Precision.HIGHEST)
    deg = dp_ref[0, :, 0] + dp_ref[1, :, 0] + 1.0
    dinv = _rsqrt(deg)
    qt = p * dinv[:, None]
    qt_ref[...] = jnp.concatenate(
        [qt, jnp.zeros((qt.shape[0], F - 3), jnp.float32)], axis=1)


_RF = N_PAD * F // 128
_BF = _RF // 2


def _dense_mid_flat(k, up_ref, qt_ref, dp_ref, V_ref, out_ref):
    t = up_ref[0] + up_ref[1] + qt_ref[...]
    deg = dp_ref[0] + dp_ref[1] + 1.0
    dinv = _rsqrt(deg)
    vk = V_ref[k - 1]
    vpat = jnp.tile(jnp.concatenate([vk, jnp.zeros((F - 3,), jnp.float32)]),
                    128 // F)
    out_ref[...] = dinv * dinv * t + dinv * vpat[None, :]


def _dense_mid_body(k, up_ref, qt_ref, dp_ref, V_ref, out_ref):
    t = up_ref[0] + up_ref[1] + qt_ref[...]
    deg = dp_ref[0, :, 0] + dp_ref[1, :, 0] + 1.0
    dinv = _rsqrt(deg)
    vk = V_ref[k - 1]
    vk16 = jnp.concatenate([vk, jnp.zeros((F - 3,), jnp.float32)])
    out_ref[...] = (dinv * dinv)[:, None] * t + dinv[:, None] * vk16[None, :]


def _dense_final_body(up_ref, qt_ref, dp_ref, V_ref, out_ref):
    t = up_ref[0] + up_ref[1] + qt_ref[...]
    deg = dp_ref[0, :, 0] + dp_ref[1, :, 0] + 1.0
    dinv = _rsqrt(deg)
    out_ref[...] = dinv[:, None] * t[:, :3] + V_ref[4][None, :]


def _full(shape):
    return pl.BlockSpec(shape, lambda i: (0,) * len(shape))


_ROWBLK = pl.BlockSpec((_BN, F), lambda i: (i, 0))
_DPBLK = pl.BlockSpec((NC, _BN, F), lambda i: (0, i, 0))
_GRID = (N_NODES // _BN,)


def kernel(x, edge_index, W1, b1, W2, b2, W3, b3, W4, b4, W5, b5):
    src = edge_index[0].astype(jnp.int32)
    dst = edge_index[1].astype(jnp.int32)
    pad = jnp.full((E_PAD - N_EDGES,), DUMMY, dtype=jnp.int32)
    src_r = jnp.concatenate([src, pad]).reshape(R_TOTAL, 128)
    dst_r = jnp.concatenate([dst, pad]).reshape(R_TOTAL, 128)
    zeros_pad = jnp.zeros((N_PAD, F), jnp.float32)
    ones_tab = jnp.ones((N_PAD, F), jnp.float32)

    sc_pass = _make_sc_pass(True)
    sc_count = _make_sc_pass(False)

    dp = sc_count(ones_tab, src_r, dst_r, zeros_pad)

    qt, V = pl.pallas_call(
        _dense0_body,
        grid=_GRID,
        in_specs=[pl.BlockSpec((_BN, 8), lambda i: (i, 0)), _DPBLK,
                  _full((8, 32)), _full((32,)),
                  _full((32, 32)), _full((32,)),
                  _full((32, 32)), _full((32,)),
                  _full((32, 32)), _full((32,)),
                  _full((32, 3)), _full((3,))],
        out_specs=[_ROWBLK, _full((5, 3))],
        out_shape=[jax.ShapeDtypeStruct((N_PAD, F), jnp.float32),
                   jax.ShapeDtypeStruct((5, 3), jnp.float32)],
    )(x, dp, W1, b1, W2, b2, W3, b3, W4, b4, W5, b5)

    dp_f = dp.reshape(NC, _RF, 128)
    fblk = pl.BlockSpec((_BF, 128), lambda i: (i, 0))
    fblk2 = pl.BlockSpec((NC, _BF, 128), lambda i: (0, i, 0))
    for k in (1, 2, 3, 4):
        up = sc_pass(qt, src_r, dst_r, zeros_pad)
        qt_f = pl.pallas_call(
            functools.partial(_dense_mid_flat, k),
            grid=(_RF // _BF,),
            in_specs=[fblk2, fblk, fblk2, _full((5, 3))],
            out_specs=fblk,
            out_shape=jax.ShapeDtypeStruct((_RF, 128), jnp.float32),
        )(up.reshape(NC, _RF, 128), qt.reshape(_RF, 128), dp_f, V)
        qt = qt_f.reshape(N_PAD, F)

    up = sc_pass(qt, src_r, dst_r, zeros_pad)
    out = pl.pallas_call(
        _dense_final_body,
        grid=_GRID,
        in_specs=[_DPBLK, _ROWBLK, _DPBLK, _full((5, 3))],
        out_specs=pl.BlockSpec((_BN, 3), lambda i: (i, 0)),
        out_shape=jax.ShapeDtypeStruct((N_NODES, 3), jnp.float32),
    )(up, qt, dp, V)
    return out


# speedup vs baseline: 82.5834x; 1.0588x over previous
import functools

import jax
import jax.numpy as jnp
from jax import lax
from jax.experimental import pallas as pl
from jax.experimental.pallas import tpu as pltpu
from jax.experimental.pallas import tpu_sc as plsc

N_NODES = 100000
N_EDGES = 3200000
F = 8
NC, NS = 2, 16
N_TILES = NC * NS
RB = 8
R_PER_TILE = 800
R_TOTAL = N_TILES * R_PER_TILE
E_PAD = R_TOTAL * 128
N_PAD = 100096
DUMMY = N_NODES

_mesh = plsc.VectorSubcoreMesh(core_axis_name="c", subcore_axis_name="s")


KB = 5


def _sc_pass(do_gather, table_hbm, srcr_hbm, dstr_hbm, zeros_hbm, out_hbm,
             acc, tab, sbuf, dbuf, rows_a, rows_b, semg, sems):
    cid = lax.axis_index("c")
    sid = lax.axis_index("s")
    w = sid * NC + cid

    zrows = N_PAD // NS
    pltpu.sync_copy(zeros_hbm.at[pl.ds(sid * zrows, zrows)],
                    acc.at[pl.ds(sid * zrows, zrows)])
    if do_gather:
        pltpu.sync_copy(table_hbm.at[pl.ds(sid * zrows, zrows)],
                        tab.at[pl.ds(sid * zrows, zrows)])
    if not do_gather:
        pltpu.sync_copy(table_hbm.at[pl.ds(0, RB * 128)], rows_a)
        pltpu.sync_copy(table_hbm.at[pl.ds(0, RB * 128)], rows_b)
    plsc.subcore_barrier()

    base = w * R_PER_TILE

    @pl.loop(0, R_PER_TILE, step=KB * RB)
    def _(r0):
        if do_gather:
            pltpu.sync_copy(srcr_hbm.at[pl.ds(base + r0, KB * RB)], sbuf)
        pltpu.sync_copy(dstr_hbm.at[pl.ds(base + r0, KB * RB)], dbuf)
        pending = [None, None]
        for k in range(KB):
            rows = rows_a if k % 2 == 0 else rows_b
            if pending[k % 2] is not None:
                for d in pending[k % 2]:
                    d.wait()
            if do_gather:
                gs = [pltpu.async_copy(tab.at[sbuf.at[k * RB + j]],
                                       rows.at[pl.ds(j * 128, 128)], semg)
                      for j in range(RB)]
                for d in gs:
                    d.wait()
            pending[k % 2] = [
                pltpu.async_copy(rows.at[pl.ds(j * 128, 128)],
                                 acc.at[dbuf.at[k * RB + j]], sems, add=True)
                for j in range(RB)]
        for ds_ in pending:
            if ds_ is not None:
                for d in ds_:
                    d.wait()

    plsc.subcore_barrier()
    pltpu.sync_copy(acc.at[pl.ds(sid * zrows, zrows)],
                    out_hbm.at[cid].at[pl.ds(sid * zrows, zrows)])


def _make_sc_pass(do_gather):
    return functools.partial(
        pl.kernel,
        out_type=jax.ShapeDtypeStruct((NC, N_PAD, F), jnp.float32),
        mesh=_mesh,
        scratch_types=[
            pltpu.VMEM_SHARED((N_PAD, F), jnp.float32),
            pltpu.VMEM_SHARED((N_PAD, F), jnp.float32),
            pltpu.VMEM((KB * RB, 128), jnp.int32),
            pltpu.VMEM((KB * RB, 128), jnp.int32),
            pltpu.VMEM((RB * 128, F), jnp.float32),
            pltpu.VMEM((RB * 128, F), jnp.float32),
            pltpu.SemaphoreType.DMA,
            pltpu.SemaphoreType.DMA,
        ],
        compiler_params=pltpu.CompilerParams(use_tc_tiling_on_sc=False),
    )(functools.partial(_sc_pass, do_gather))


_BN = 2000


def _rsqrt(x):
    y = lax.rsqrt(x)
    y = y * (1.5 - 0.5 * x * y * y)
    y = y * (1.5 - 0.5 * x * y * y)
    return y


def _dense0_body(x_ref, dp_ref, W1, b1, W2, b2, W3, b3, W4, b4, W5, b5,
                 qt_ref, V_ref):
    C5 = W5[...]
    C4 = jnp.dot(W4[...], C5, preferred_element_type=jnp.float32, precision=lax.Precision.HIGHEST)
    C3 = jnp.dot(W3[...], C4, preferred_element_type=jnp.float32, precision=lax.Precision.HIGHEST)
    C2 = jnp.dot(W2[...], C3, preferred_element_type=jnp.float32, precision=lax.Precision.HIGHEST)
    Wall = jnp.dot(W1[...], C2, preferred_element_type=jnp.float32, precision=lax.Precision.HIGHEST)
    v1 = jnp.dot(b1[...], C2, preferred_element_type=jnp.float32, precision=lax.Precision.HIGHEST)
    v2 = jnp.dot(b2[...], C3, preferred_element_type=jnp.float32, precision=lax.Precision.HIGHEST)
    v3 = jnp.dot(b3[...], C4, preferred_element_type=jnp.float32, precision=lax.Precision.HIGHEST)
    v4 = jnp.dot(b4[...], C5, preferred_element_type=jnp.float32, precision=lax.Precision.HIGHEST)
    v5 = b5[...]
    V_ref[...] = jnp.stack([v1, v2, v3, v4, v5])

    p = jnp.dot(x_ref[...], Wall, preferred_element_type=jnp.float32, precision=lax.Precision.HIGHEST)
    deg = dp_ref[0, :, 0] + dp_ref[1, :, 0] + 1.0
    dinv = _rsqrt(deg)
    qt = p * dinv[:, None]
    qt_ref[...] = jnp.concatenate(
        [qt, jnp.zeros((qt.shape[0], F - 3), jnp.float32)], axis=1)


_RF = N_PAD * F // 128
_BF = _RF // 2


def _dense_mid_flat(k, up_ref, qt_ref, dp_ref, V_ref, out_ref):
    t = up_ref[0] + up_ref[1] + qt_ref[...]
    deg = dp_ref[0] + dp_ref[1] + 1.0
    dinv = _rsqrt(deg)
    vk = V_ref[k - 1]
    vpat = jnp.tile(jnp.concatenate([vk, jnp.zeros((F - 3,), jnp.float32)]),
                    128 // F)
    out_ref[...] = dinv * dinv * t + dinv * vpat[None, :]


def _dense_mid_body(k, up_ref, qt_ref, dp_ref, V_ref, out_ref):
    t = up_ref[0] + up_ref[1] + qt_ref[...]
    deg = dp_ref[0, :, 0] + dp_ref[1, :, 0] + 1.0
    dinv = _rsqrt(deg)
    vk = V_ref[k - 1]
    vk16 = jnp.concatenate([vk, jnp.zeros((F - 3,), jnp.float32)])
    out_ref[...] = (dinv * dinv)[:, None] * t + dinv[:, None] * vk16[None, :]


def _dense_final_flat(up_ref, qt_ref, dp_ref, V_ref, out_ref):
    t = up_ref[0] + up_ref[1] + qt_ref[...]
    deg = dp_ref[0] + dp_ref[1] + 1.0
    dinv = _rsqrt(deg)
    vpat = jnp.tile(jnp.concatenate([V_ref[4], jnp.zeros((F - 3,), jnp.float32)]),
                    128 // F)
    out_ref[...] = dinv * t + vpat[None, :]


def _dense_final_body(up_ref, qt_ref, dp_ref, V_ref, out_ref):
    t = up_ref[0] + up_ref[1] + qt_ref[...]
    deg = dp_ref[0, :, 0] + dp_ref[1, :, 0] + 1.0
    dinv = _rsqrt(deg)
    out_ref[...] = dinv[:, None] * t[:, :3] + V_ref[4][None, :]


def _full(shape):
    return pl.BlockSpec(shape, lambda i: (0,) * len(shape))


_ROWBLK = pl.BlockSpec((_BN, F), lambda i: (i, 0))
_DPBLK = pl.BlockSpec((NC, _BN, F), lambda i: (0, i, 0))
_GRID = (N_NODES // _BN,)


def kernel(x, edge_index, W1, b1, W2, b2, W3, b3, W4, b4, W5, b5):
    src = edge_index[0].astype(jnp.int32)
    dst = edge_index[1].astype(jnp.int32)
    pad = jnp.full((E_PAD - N_EDGES,), DUMMY, dtype=jnp.int32)
    src_r = jnp.concatenate([src, pad]).reshape(R_TOTAL, 128)
    dst_r = jnp.concatenate([dst, pad]).reshape(R_TOTAL, 128)
    zeros_pad = jnp.zeros((N_PAD, F), jnp.float32)
    ones_tab = jnp.ones((N_PAD, F), jnp.float32)

    sc_pass = _make_sc_pass(True)
    sc_count = _make_sc_pass(False)

    dp = sc_count(ones_tab, src_r, dst_r, zeros_pad)

    qt, V = pl.pallas_call(
        _dense0_body,
        grid=_GRID,
        in_specs=[pl.BlockSpec((_BN, 8), lambda i: (i, 0)), _DPBLK,
                  _full((8, 32)), _full((32,)),
                  _full((32, 32)), _full((32,)),
                  _full((32, 32)), _full((32,)),
                  _full((32, 32)), _full((32,)),
                  _full((32, 3)), _full((3,))],
        out_specs=[_ROWBLK, _full((5, 3))],
        out_shape=[jax.ShapeDtypeStruct((N_PAD, F), jnp.float32),
                   jax.ShapeDtypeStruct((5, 3), jnp.float32)],
    )(x, dp, W1, b1, W2, b2, W3, b3, W4, b4, W5, b5)

    dp_f = dp.reshape(NC, _RF, 128)
    fblk = pl.BlockSpec((_BF, 128), lambda i: (i, 0))
    fblk2 = pl.BlockSpec((NC, _BF, 128), lambda i: (0, i, 0))
    for k in (1, 2, 3, 4):
        up = sc_pass(qt, src_r, dst_r, zeros_pad)
        qt_f = pl.pallas_call(
            functools.partial(_dense_mid_flat, k),
            grid=(_RF // _BF,),
            in_specs=[fblk2, fblk, fblk2, _full((5, 3))],
            out_specs=fblk,
            out_shape=jax.ShapeDtypeStruct((_RF, 128), jnp.float32),
        )(up.reshape(NC, _RF, 128), qt.reshape(_RF, 128), dp_f, V)
        qt = qt_f.reshape(N_PAD, F)

    up = sc_pass(qt, src_r, dst_r, zeros_pad)
    out_f = pl.pallas_call(
        _dense_final_flat,
        grid=(_RF // _BF,),
        in_specs=[fblk2, fblk, fblk2, _full((5, 3))],
        out_specs=fblk,
        out_shape=jax.ShapeDtypeStruct((_RF, 128), jnp.float32),
    )(up.reshape(NC, _RF, 128), qt.reshape(_RF, 128), dp_f, V)
    return out_f.reshape(N_PAD, F)[:N_NODES, :3]
